# Initial kernel scaffold; baseline (speedup 1.0000x reference)
import jax
import jax.numpy as jnp
from jax.experimental import pallas as pl


def kernel(updates, mask):
    raise NotImplementedError("write your pallas kernel here")


# speedup vs baseline: 14.1260x; 14.1260x over previous
import functools

import jax
import jax.numpy as jnp
from jax import lax
from jax.experimental import pallas as pl
from jax.experimental.pallas import tpu as pltpu
from jax.experimental.pallas import tpu_sc as plsc

B, H, W, C = 4, 112, 112, 192
PIX = H * W
H_OUT, W_OUT = 2 * H, 2 * W
PIX_OUT = H_OUT * W_OUT
NCOL = B * C

NC, NS = 2, 16
NW = NC * NS
COLS_PER = NCOL // NW
STAGE = 3136
NSTAGE = PIX // STAGE
CHUNKS = STAGE // 16


BLK_A = 896


def _pre_body(upd_ref, mask_ref, valt_ref, pt_ref):
  v = upd_ref[0]
  m = mask_ref[0]
  valt_ref[...] = v.T
  pt_ref[...] = (m // C).T


@jax.jit
def _tc_pre(updates, mask):
  return pl.pallas_call(
      _pre_body,
      grid=(B, PIX // BLK_A),
      in_specs=[
          pl.BlockSpec((1, BLK_A, C), lambda b, i: (b, i, 0)),
          pl.BlockSpec((1, BLK_A, C), lambda b, i: (b, i, 0)),
      ],
      out_specs=[
          pl.BlockSpec((C, BLK_A), lambda b, i: (b, i)),
          pl.BlockSpec((C, BLK_A), lambda b, i: (b, i)),
      ],
      out_shape=[
          jax.ShapeDtypeStruct((NCOL, PIX), jnp.float32),
          jax.ShapeDtypeStruct((NCOL, PIX), jnp.int32),
      ],
  )(updates, mask)


@functools.partial(
    pl.kernel,
    out_type=jax.ShapeDtypeStruct((NCOL, PIX_OUT), jnp.float32),
    mesh=plsc.VectorSubcoreMesh(core_axis_name="c", subcore_axis_name="s"),
    scratch_types=[
        pltpu.VMEM((PIX_OUT,), jnp.float32),
        pltpu.VMEM((PIX_OUT,), jnp.int32),
        pltpu.VMEM((STAGE,), jnp.float32),
        pltpu.VMEM((STAGE,), jnp.int32),
    ],
    compiler_params=pltpu.CompilerParams(
        use_tc_tiling_on_sc=False, needs_layout_passes=False),
)
def _sc_scatter(valt_hbm, pt_hbm, out_hbm, acc, chk, vstage, pstage):
  wid = lax.axis_index("c") * NS + lax.axis_index("s")
  lanes = lax.iota(jnp.int32, 16)
  zeros16 = jnp.zeros((16,), jnp.float32)

  def per_col(j, carry):
    col = wid * COLS_PER + j

    def zero_body(z, c2):
      base = z * 128
      for u in range(8):
        acc[pl.ds(base + u * 16, 16)] = zeros16
      return c2

    lax.fori_loop(0, PIX_OUT // 128, zero_body, 0)

    def per_stage(s, c2):
      pltpu.sync_copy(valt_hbm.at[col, pl.ds(s * STAGE, STAGE)], vstage)
      pltpu.sync_copy(pt_hbm.at[col, pl.ds(s * STAGE, STAGE)], pstage)

      def per_chunk(i, c3):
        idx = pstage[pl.ds(i * 16, 16)]
        val = vstage[pl.ds(i * 16, 16)]
        plsc.store_scatter(chk, [idx], lanes)
        got = plsc.load_gather(chk, [idx])
        has_dup = jnp.any(got != lanes)

        @pl.when(jnp.logical_not(has_dup))
        def _fast():
          plsc.addupdate_scatter(acc, [idx], val)

        @pl.when(has_dup)
        def _slow():
          for l in range(16):
            plsc.addupdate_scatter(acc, [idx], val, mask=lanes == l)

        return c3

      lax.fori_loop(0, CHUNKS, per_chunk, 0)
      return c2

    lax.fori_loop(0, NSTAGE, per_stage, 0)
    pltpu.sync_copy(acc, out_hbm.at[col])
    return carry

  lax.fori_loop(0, COLS_PER, per_col, 0)


BLK_C = 896


def _post_body(in_ref, out_ref):
  out_ref[0] = in_ref[0].T


@jax.jit
def _tc_post(outt):
  return pl.pallas_call(
      _post_body,
      grid=(B, PIX_OUT // BLK_C),
      in_specs=[pl.BlockSpec((1, C, BLK_C), lambda b, i: (b, 0, i))],
      out_specs=pl.BlockSpec((1, BLK_C, C), lambda b, i: (b, i, 0)),
      out_shape=jax.ShapeDtypeStruct((B, PIX_OUT, C), jnp.float32),
  )(outt)


def kernel(updates, mask):
  u = updates.reshape(B, PIX, C)
  m = mask.reshape(B, PIX, C).astype(jnp.int32)
  valt, pt = _tc_pre(u, m)
  outt = _sc_scatter(valt, pt)
  out = _tc_post(outt.reshape(B, C, PIX_OUT))
  return out.reshape(B, H_OUT, W_OUT, C)


# speedup vs baseline: 18.5770x; 1.3151x over previous
import functools

import jax
import jax.numpy as jnp
from jax import lax
from jax.experimental import pallas as pl
from jax.experimental.pallas import tpu as pltpu
from jax.experimental.pallas import tpu_sc as plsc

B, H, W, C = 4, 112, 112, 192
PIX = H * W
H_OUT, W_OUT = 2 * H, 2 * W
PIX_OUT = H_OUT * W_OUT
NCOL = B * C

NC, NS = 2, 16
NW = NC * NS
COLS_PER = NCOL // NW
ROWS_IN = 104
ROWS_VALID = PIX // 128
ROWS_OUT = PIX_OUT // 128


def _pre_body(upd_ref, mask_ref, valt_ref, pt_ref):
  v = upd_ref[0]
  m = mask_ref[0]
  t = pl.program_id(1)
  row = lax.broadcasted_iota(jnp.int32, (1024, C), 0)
  valid = (t * 1024 + row) < PIX
  v = jnp.where(valid, v, 0.0)
  p = jnp.where(valid, m // C, 0)
  valt_ref[...] = v.T.reshape(C, 8, 128)
  pt_ref[...] = p.T.reshape(C, 8, 128)


@jax.jit
def _tc_pre(updates, mask):
  return pl.pallas_call(
      _pre_body,
      grid=(B, ROWS_IN // 8),
      in_specs=[
          pl.BlockSpec((1, 1024, C), lambda b, t: (b, t, 0)),
          pl.BlockSpec((1, 1024, C), lambda b, t: (b, t, 0)),
      ],
      out_specs=[
          pl.BlockSpec((C, 8, 128), lambda b, t: (b, t, 0)),
          pl.BlockSpec((C, 8, 128), lambda b, t: (b, t, 0)),
      ],
      out_shape=[
          jax.ShapeDtypeStruct((NCOL, ROWS_IN, 128), jnp.float32),
          jax.ShapeDtypeStruct((NCOL, ROWS_IN, 128), jnp.int32),
      ],
  )(updates, mask)


def _gather16(x, idx):
  return lax.gather(
      x, idx[:, None],
      lax.GatherDimensionNumbers(
          offset_dims=(), collapsed_slice_dims=(0,), start_index_map=(0,)),
      slice_sizes=(1,), mode=lax.GatherScatterMode.PROMISE_IN_BOUNDS)


@functools.partial(
    pl.kernel,
    out_type=jax.ShapeDtypeStruct((NCOL, ROWS_OUT, 128), jnp.float32),
    mesh=plsc.VectorSubcoreMesh(core_axis_name="c", subcore_axis_name="s"),
    scratch_types=[
        pltpu.VMEM((ROWS_OUT, 128), jnp.float32),
        pltpu.VMEM((ROWS_IN, 128), jnp.float32),
        pltpu.VMEM((ROWS_IN, 128), jnp.int32),
    ],
    compiler_params=pltpu.CompilerParams(needs_layout_passes=False),
)
def _sc_scatter(valt_hbm, pt_hbm, out_hbm, acc, vstage, pstage):
  wid = lax.axis_index("c") * NS + lax.axis_index("s")
  iota = lax.iota(jnp.int32, 16)
  iota1 = iota + 1
  idx_p1 = jnp.minimum(iota1, 15)
  idx_m1 = jnp.maximum(iota - 1, 0)
  is0 = iota == 0
  is15 = iota == 15
  zeros16 = jnp.zeros((16,), jnp.float32)

  def per_col(j, carry):
    col = wid * COLS_PER + j

    def zero_body(r, c2):
      for u in range(8):
        acc[r, pl.ds(u * 16, 16)] = zeros16
      return c2

    lax.fori_loop(0, ROWS_OUT, zero_body, 0)

    pltpu.sync_copy(valt_hbm.at[col], vstage)
    pltpu.sync_copy(pt_hbm.at[col], pstage)

    def row_body(r, c2):
      for cc in range(8):
        p = pstage[r, pl.ds(cc * 16, 16)]
        v = vstage[r, pl.ds(cc * 16, 16)]
        sp, sv = plsc.sort_key_val(p, v)
        nxt = _gather16(sp, idx_p1)
        is_last = (sp != nxt) | is15
        cs = plsc.cumsum(sv)
        markp1 = jnp.where(is_last, iota1, 0)
        em = jnp.where(is0, 0, _gather16(markp1, idx_m1))
        bp = plsc.cummax(em)
        pcs = _gather16(cs, jnp.maximum(bp - 1, 0))
        seg = cs - jnp.where(bp > 0, pcs, 0.0)
        hi = lax.shift_right_logical(sp, 7)
        lo = sp & 127
        plsc.addupdate_scatter(acc, [hi, lo], seg, mask=is_last)
      return c2

    lax.fori_loop(0, ROWS_IN, row_body, 0)
    pltpu.sync_copy(acc, out_hbm.at[col])
    return carry

  lax.fori_loop(0, COLS_PER, per_col, 0)


def _post_body(in_ref, out_ref):
  x = in_ref[...]
  out_ref[0] = x.reshape(C, 56 * 128).T


@jax.jit
def _tc_post(outt):
  return pl.pallas_call(
      _post_body,
      grid=(B, ROWS_OUT // 56),
      in_specs=[pl.BlockSpec((C, 56, 128), lambda b, t: (b, t, 0))],
      out_specs=pl.BlockSpec((1, 56 * 128, C), lambda b, t: (b, t, 0)),
      out_shape=jax.ShapeDtypeStruct((B, PIX_OUT, C), jnp.float32),
  )(outt)


def kernel(updates, mask):
  u = updates.reshape(B, PIX, C)
  m = mask.reshape(B, PIX, C).astype(jnp.int32)
  valt, pt = _tc_pre(u, m)
  out = _tc_post(_sc_scatter(valt, pt))
  return out.reshape(B, H_OUT, W_OUT, C)
